# baseline (speedup 1.0000x reference)
import functools

import jax
import jax.numpy as jnp
from jax.experimental import pallas as pl


def _cumsum_lanes(x, n):
    sh = 1
    while sh < n:
        x = x + jnp.concatenate(
            [jnp.zeros_like(x[..., :sh]), x[..., :n - sh]], axis=-1)
        sh *= 2
    return x


def _fps_body(xyzT_ref, newxyz_ref, *, n, s_count):
    x = xyzT_ref[:, 0, :]
    y = xyzT_ref[:, 1, :]
    z = xyzT_ref[:, 2, :]
    b = x.shape[0]
    idx_row = jax.lax.broadcasted_iota(jnp.int32, (1, n), 1)

    def fps_step(i, carry):
        dists, far = carry
        sel = jnp.where(idx_row == far, 1.0, 0.0)
        cx = jnp.sum(x * sel, axis=1, keepdims=True)
        cy = jnp.sum(y * sel, axis=1, keepdims=True)
        cz = jnp.sum(z * sel, axis=1, keepdims=True)
        row = jnp.concatenate(
            [cx[:, :, None], cy[:, :, None], cz[:, :, None]], axis=2)
        newxyz_ref[:, pl.ds(i, 1), :] = row
        dx = x - cx
        dy = y - cy
        dz = z - cz
        d = dx * dx + dy * dy + dz * dz
        dists = jnp.minimum(dists, d)
        m = jnp.max(dists, axis=1, keepdims=True)
        far = jnp.min(jnp.where(dists == m, idx_row, n),
                      axis=1, keepdims=True).astype(jnp.int32)
        return dists, far

    init = (jnp.full((b, n), 1e10, jnp.float32), jnp.zeros((b, 1), jnp.int32))
    jax.lax.fori_loop(0, s_count, fps_step, init)


def _fps_vec(x, y, z, s_count, n):
    b = x.shape[0]
    idx_row = jax.lax.broadcasted_iota(jnp.int32, (1, n), 1)
    s_io = jax.lax.broadcasted_iota(jnp.int32, (1, s_count), 1)

    def fps_step(i, carry):
        dists, far, cxs, cys, czs = carry
        sel = jnp.where(idx_row == far, 1.0, 0.0)
        cx = jnp.sum(x * sel, axis=1, keepdims=True)
        cy = jnp.sum(y * sel, axis=1, keepdims=True)
        cz = jnp.sum(z * sel, axis=1, keepdims=True)
        cxs = jnp.where(s_io == i, cx, cxs)
        cys = jnp.where(s_io == i, cy, cys)
        czs = jnp.where(s_io == i, cz, czs)
        dx = x - cx
        dy = y - cy
        dz = z - cz
        d = dx * dx + dy * dy + dz * dz
        dists = jnp.minimum(dists, d)
        m = jnp.max(dists, axis=1, keepdims=True)
        far = jnp.min(jnp.where(dists == m, idx_row, n),
                      axis=1, keepdims=True).astype(jnp.int32)
        return dists, far, cxs, cys, czs

    init = (jnp.full((b, n), 1e10, jnp.float32),
            jnp.zeros((b, 1), jnp.int32),
            jnp.zeros((b, s_count), jnp.float32),
            jnp.zeros((b, s_count), jnp.float32),
            jnp.zeros((b, s_count), jnp.float32))
    _, _, cxs, cys, czs = jax.lax.fori_loop(0, s_count, fps_step, init)
    return cxs, cys, czs


def _slot_onehot(d2, inb, n, k_count):
    rank = _cumsum_lanes(jnp.where(inb, 1.0, 0.0), n)
    count = rank[..., n - 1:n]
    kio_shape = (1,) * (d2.ndim - 1) + (k_count,)
    kio = jax.lax.broadcasted_iota(jnp.int32, kio_shape,
                                   d2.ndim - 1).astype(jnp.float32)
    tgt = jnp.where(count > kio, kio + 1.0, 1.0)
    return jnp.where((rank[..., None, :] == tgt[..., :, None])
                     & inb[..., None, :], 1.0, 0.0)


def _split_operand(obs):
    xyz_hi = obs[..., 0:3].astype(jnp.bfloat16).astype(jnp.float32)
    xyz_lo = obs[..., 0:3] - xyz_hi
    return jnp.concatenate([obs, xyz_lo], axis=-1)


def _geom_body(newxyz_in_ref, xyzT_ref, operand_ref, w1_ref, pre_ref,
               *, n, s_count, k_count, radius, cf):
    centers = newxyz_in_ref[0]
    sx = centers[:, 0:1]
    sy = centers[:, 1:2]
    sz = centers[:, 2:3]
    dx = sx - xyzT_ref[0, 0:1, :]
    dy = sy - xyzT_ref[0, 1:2, :]
    dz = sz - xyzT_ref[0, 2:3, :]
    d2 = dx * dx + dy * dy + dz * dz
    inb = d2 < (radius * radius)
    oh = _slot_onehot(d2, inb, n, k_count).reshape(s_count * k_count, n)
    oh16 = oh.astype(jnp.bfloat16)
    g = jnp.dot(oh16, operand_ref[0],
                preferred_element_type=jnp.float32)
    g3 = g.reshape(s_count, k_count, -1)
    gxyz = g3[:, :, 0:3] + g3[:, :, cf:cf + 3]
    gx = (gxyz - centers[:, None, :]) / radius
    x = jnp.concatenate([gx, g3[:, :, 3:cf]], axis=-1)
    xf = x.reshape(s_count * k_count, -1)
    pre = jnp.dot(xf, w1_ref[...], preferred_element_type=jnp.float32)
    pre_ref[0] = pre.reshape(s_count, k_count, -1)


def _bq_group(x_pl, y_pl, z_pl, cxs, cys, czs, obs, radius, k_count):
    b, n = x_pl.shape
    s = cxs.shape[1]
    cf = obs.shape[-1]
    dx = cxs[:, :, None] - x_pl[:, None, :]
    dy = cys[:, :, None] - y_pl[:, None, :]
    dz = czs[:, :, None] - z_pl[:, None, :]
    d2 = dx * dx + dy * dy + dz * dz
    inb = d2 < (radius * radius)
    oh = _slot_onehot(d2, inb, n, k_count).reshape(b, s * k_count, n)
    operand = _split_operand(obs)
    g = jnp.stack([jnp.dot(oh[i], operand[i],
                           preferred_element_type=jnp.float32)
                   for i in range(b)], axis=0)
    g4 = g.reshape(b, s, k_count, -1)
    gxyz = g4[..., 0:3] + g4[..., cf:cf + 3]
    centers = jnp.concatenate(
        [cxs[:, :, None], cys[:, :, None], czs[:, :, None]], axis=2)
    gx = (gxyz - centers[:, :, None, :]) / radius
    x = jnp.concatenate([gx, g4[..., 3:cf]], axis=-1)
    return x.reshape(b * s * k_count, cf)


def _bn_relu(x, g, bb):
    m = jnp.mean(x, axis=0, keepdims=True)
    v = jnp.mean((x - m) ** 2, axis=0, keepdims=True)
    return jax.nn.relu((x - m) * jax.lax.rsqrt(v + 1e-5) * g + bb)


def _mlp_chain(xf, w1, g1, b1, w2, g2, b2, w3, g3, b3):
    x = jnp.dot(xf, w1, preferred_element_type=jnp.float32)
    x = _bn_relu(x, g1, b1)
    x = jnp.dot(x, w2, preferred_element_type=jnp.float32)
    x = _bn_relu(x, g2, b2)
    x = jnp.dot(x, w3, preferred_element_type=jnp.float32)
    return _bn_relu(x, g3, b3)


def _mlp_body(pre_ref, g1_ref, b1_ref, w2_ref, g2_ref, b2_ref, w3_ref,
              g3_ref, b3_ref, out_ref, *, b, s_count, k_count):
    c1 = pre_ref.shape[-1]
    x = pre_ref[...].reshape(b * s_count * k_count, c1)
    x = _bn_relu(x, g1_ref[...], b1_ref[...])
    x = jnp.dot(x, w2_ref[...], preferred_element_type=jnp.float32)
    x = _bn_relu(x, g2_ref[...], b2_ref[...])
    x = jnp.dot(x, w3_ref[...], preferred_element_type=jnp.float32)
    x = _bn_relu(x, g3_ref[...], b3_ref[...])
    pooled = jnp.max(x.reshape(b * s_count, k_count, -1), axis=1)
    out_ref[...] = pooled.reshape(b, s_count, -1)


def _tail_body(obs2_ref, xyzT2_ref,
               w21_ref, g21_ref, b21_ref, w22_ref, g22_ref, b22_ref,
               w23_ref, g23_ref, b23_ref,
               w31_ref, g31_ref, b31_ref, w32_ref, g32_ref, b32_ref,
               w33_ref, g33_ref, b33_ref,
               mask_ref, out_ref, *, s2, k2, r2, k3, r3):
    obs2 = obs2_ref[...]
    b, n2, _ = obs2.shape
    x_pl = xyzT2_ref[:, 0, :]
    y_pl = xyzT2_ref[:, 1, :]
    z_pl = xyzT2_ref[:, 2, :]

    cxs, cys, czs = _fps_vec(x_pl, y_pl, z_pl, s2, n2)
    x16 = _bq_group(x_pl, y_pl, z_pl, cxs, cys, czs, obs2, r2, k2)
    x = _mlp_chain(x16, w21_ref[...], g21_ref[...], b21_ref[...],
                   w22_ref[...], g22_ref[...], b22_ref[...],
                   w23_ref[...], g23_ref[...], b23_ref[...])
    feats2 = jnp.max(x.reshape(b * s2, k2, -1), axis=1)
    feats2 = feats2.reshape(b, s2, -1)

    newxyz2 = jnp.concatenate(
        [cxs[:, :, None], cys[:, :, None], czs[:, :, None]], axis=2)
    obs3 = jnp.concatenate([newxyz2, feats2], axis=-1)
    x16 = _bq_group(cxs, cys, czs,
                    cxs[:, 0:1], cys[:, 0:1], czs[:, 0:1],
                    obs3, r3, k3)
    x = _mlp_chain(x16, w31_ref[...], g31_ref[...], b31_ref[...],
                   w32_ref[...], g32_ref[...], b32_ref[...],
                   w33_ref[...], g33_ref[...], b33_ref[...])
    pooled = jnp.max(x.reshape(b, k3, -1), axis=1)
    out_ref[...] = pooled * mask_ref[...]


def kernel(observation, mask, params):
    b, n, cf = observation.shape
    xyz_t = observation[:, :, :3].transpose(0, 2, 1)
    (npoint, radius, nsample) = (64, 1.2, 16)
    lp = params[0]
    c1 = lp[0][0].shape[0]

    fps = pl.pallas_call(
        functools.partial(_fps_body, n=n, s_count=npoint),
        out_shape=jax.ShapeDtypeStruct((b, npoint, 3), jnp.float32),
    )
    newxyz = fps(xyz_t)

    obs16 = observation.astype(jnp.bfloat16)
    xyz_lo16 = (observation[:, :, 0:3]
                - obs16[:, :, 0:3].astype(jnp.float32)).astype(jnp.bfloat16)
    operand = jnp.concatenate([obs16, xyz_lo16], axis=-1)
    geom = pl.pallas_call(
        functools.partial(_geom_body, n=n, s_count=npoint,
                          k_count=nsample, radius=radius, cf=cf),
        grid=(b,),
        in_specs=[
            pl.BlockSpec((1, npoint, 3), lambda i: (i, 0, 0)),
            pl.BlockSpec((1, 3, n), lambda i: (i, 0, 0)),
            pl.BlockSpec((1, n, cf + 3), lambda i: (i, 0, 0)),
            pl.BlockSpec((cf, c1), lambda i: (0, 0)),
        ],
        out_specs=pl.BlockSpec((1, npoint, nsample, c1),
                               lambda i: (i, 0, 0, 0)),
        out_shape=jax.ShapeDtypeStruct((b, npoint, nsample, c1),
                                       jnp.float32),
    )
    pre = geom(newxyz, xyz_t, operand, lp[0][0].T)

    c3 = lp[2][0].shape[0]
    mlp = pl.pallas_call(
        functools.partial(_mlp_body, b=b, s_count=npoint, k_count=nsample),
        out_shape=jax.ShapeDtypeStruct((b, npoint, c3), jnp.float32),
    )
    feats1 = mlp(pre,
                 lp[0][1].reshape(1, -1), lp[0][2].reshape(1, -1),
                 lp[1][0].T, lp[1][1].reshape(1, -1), lp[1][2].reshape(1, -1),
                 lp[2][0].T, lp[2][1].reshape(1, -1), lp[2][2].reshape(1, -1))

    obs2 = jnp.concatenate([newxyz, feats1], axis=-1)
    xyz_t2 = newxyz.transpose(0, 2, 1)
    lp2, lp3 = params[1], params[2]
    tail = pl.pallas_call(
        functools.partial(_tail_body, s2=16, k2=16, r2=3.6, k3=16, r3=3.6),
        out_shape=jax.ShapeDtypeStruct((b, lp3[2][0].shape[0]), jnp.float32),
    )
    y = tail(obs2, xyz_t2,
             lp2[0][0].T, lp2[0][1].reshape(1, -1), lp2[0][2].reshape(1, -1),
             lp2[1][0].T, lp2[1][1].reshape(1, -1), lp2[1][2].reshape(1, -1),
             lp2[2][0].T, lp2[2][1].reshape(1, -1), lp2[2][2].reshape(1, -1),
             lp3[0][0].T, lp3[0][1].reshape(1, -1), lp3[0][2].reshape(1, -1),
             lp3[1][0].T, lp3[1][1].reshape(1, -1), lp3[1][2].reshape(1, -1),
             lp3[2][0].T, lp3[2][1].reshape(1, -1), lp3[2][2].reshape(1, -1),
             mask)
    return (y, y)


# speedup vs baseline: 1.2073x; 1.2073x over previous
import functools

import jax
import jax.numpy as jnp
from jax.experimental import pallas as pl


def _cumsum_lanes(x, n):
    sh = 1
    while sh < n:
        x = x + jnp.concatenate(
            [jnp.zeros_like(x[..., :sh]), x[..., :n - sh]], axis=-1)
        sh *= 2
    return x


def _fps_body(xyzT_ref, newxyz_ref, *, n, s_count):
    x = xyzT_ref[:, 0, :]
    y = xyzT_ref[:, 1, :]
    z = xyzT_ref[:, 2, :]
    b = x.shape[0]
    idx_row = jax.lax.broadcasted_iota(jnp.int32, (1, n), 1)

    def fps_step(i, carry):
        dists, far = carry
        sel = jnp.where(idx_row == far, 1.0, 0.0)
        cx = jnp.sum(x * sel, axis=1, keepdims=True)
        cy = jnp.sum(y * sel, axis=1, keepdims=True)
        cz = jnp.sum(z * sel, axis=1, keepdims=True)
        row = jnp.concatenate(
            [cx[:, :, None], cy[:, :, None], cz[:, :, None]], axis=2)
        newxyz_ref[:, pl.ds(i, 1), :] = row
        dx = x - cx
        dy = y - cy
        dz = z - cz
        d = dx * dx + dy * dy + dz * dz
        dists = jnp.minimum(dists, d)
        m = jnp.max(dists, axis=1, keepdims=True)
        far = jnp.min(jnp.where(dists == m, idx_row, n),
                      axis=1, keepdims=True).astype(jnp.int32)
        return dists, far

    init = (jnp.full((b, n), 1e10, jnp.float32), jnp.zeros((b, 1), jnp.int32))
    jax.lax.fori_loop(0, s_count, fps_step, init)


def _fps_vec(x, y, z, s_count, n):
    b = x.shape[0]
    idx_row = jax.lax.broadcasted_iota(jnp.int32, (1, n), 1)
    s_io = jax.lax.broadcasted_iota(jnp.int32, (1, s_count), 1)

    def fps_step(i, carry):
        dists, far, cxs, cys, czs = carry
        sel = jnp.where(idx_row == far, 1.0, 0.0)
        cx = jnp.sum(x * sel, axis=1, keepdims=True)
        cy = jnp.sum(y * sel, axis=1, keepdims=True)
        cz = jnp.sum(z * sel, axis=1, keepdims=True)
        cxs = jnp.where(s_io == i, cx, cxs)
        cys = jnp.where(s_io == i, cy, cys)
        czs = jnp.where(s_io == i, cz, czs)
        dx = x - cx
        dy = y - cy
        dz = z - cz
        d = dx * dx + dy * dy + dz * dz
        dists = jnp.minimum(dists, d)
        m = jnp.max(dists, axis=1, keepdims=True)
        far = jnp.min(jnp.where(dists == m, idx_row, n),
                      axis=1, keepdims=True).astype(jnp.int32)
        return dists, far, cxs, cys, czs

    init = (jnp.full((b, n), 1e10, jnp.float32),
            jnp.zeros((b, 1), jnp.int32),
            jnp.zeros((b, s_count), jnp.float32),
            jnp.zeros((b, s_count), jnp.float32),
            jnp.zeros((b, s_count), jnp.float32))
    _, _, cxs, cys, czs = jax.lax.fori_loop(0, s_count, fps_step, init)
    return cxs, cys, czs


def _slot_onehot(d2, inb, n, k_count):
    rank = _cumsum_lanes(jnp.where(inb, 1.0, 0.0), n)
    count = rank[..., n - 1:n]
    kio_shape = (1,) * (d2.ndim - 1) + (k_count,)
    kio = jax.lax.broadcasted_iota(jnp.int32, kio_shape,
                                   d2.ndim - 1).astype(jnp.float32)
    tgt = jnp.where(count > kio, kio + 1.0, 1.0)
    return jnp.where((rank[..., None, :] == tgt[..., :, None])
                     & inb[..., None, :], 1.0, 0.0)


def _split_operand(obs):
    xyz_hi = obs[..., 0:3].astype(jnp.bfloat16).astype(jnp.float32)
    xyz_lo = obs[..., 0:3] - xyz_hi
    return jnp.concatenate([obs, xyz_lo], axis=-1)


def _geom_body(newxyz_in_ref, xyzT_ref, obs_ref, w1_ref, pre_ref,
               *, n, s_count, k_count, radius):
    centers = newxyz_in_ref[0]
    sx = centers[:, 0:1]
    sy = centers[:, 1:2]
    sz = centers[:, 2:3]
    dx = sx - xyzT_ref[0, 0:1, :]
    dy = sy - xyzT_ref[0, 1:2, :]
    dz = sz - xyzT_ref[0, 2:3, :]
    d2 = dx * dx + dy * dy + dz * dz
    inb = d2 < (radius * radius)
    oh16 = _slot_onehot(d2, inb, n, k_count).reshape(s_count * k_count, n)

    cf = w1_ref.shape[0]
    operand = _split_operand(obs_ref[0])
    g = jnp.dot(oh16, operand,
                preferred_element_type=jnp.float32)
    g3 = g.reshape(s_count, k_count, -1)
    gxyz = g3[:, :, 0:3] + g3[:, :, cf:cf + 3]
    gx = (gxyz - centers[:, None, :]) / radius
    x = jnp.concatenate([gx, g3[:, :, 3:cf]], axis=-1)
    xf = x.reshape(s_count * k_count, -1)
    pre = jnp.dot(xf, w1_ref[...], preferred_element_type=jnp.float32)
    pre_ref[0] = pre.reshape(s_count, k_count, -1)


def _bq_group(x_pl, y_pl, z_pl, cxs, cys, czs, obs, radius, k_count):
    b, n = x_pl.shape
    s = cxs.shape[1]
    cf = obs.shape[-1]
    dx = cxs[:, :, None] - x_pl[:, None, :]
    dy = cys[:, :, None] - y_pl[:, None, :]
    dz = czs[:, :, None] - z_pl[:, None, :]
    d2 = dx * dx + dy * dy + dz * dz
    inb = d2 < (radius * radius)
    oh = _slot_onehot(d2, inb, n, k_count).reshape(b, s * k_count, n)
    operand = _split_operand(obs)
    g = jnp.stack([jnp.dot(oh[i], operand[i],
                           preferred_element_type=jnp.float32)
                   for i in range(b)], axis=0)
    g4 = g.reshape(b, s, k_count, -1)
    gxyz = g4[..., 0:3] + g4[..., cf:cf + 3]
    centers = jnp.concatenate(
        [cxs[:, :, None], cys[:, :, None], czs[:, :, None]], axis=2)
    gx = (gxyz - centers[:, :, None, :]) / radius
    x = jnp.concatenate([gx, g4[..., 3:cf]], axis=-1)
    return x.reshape(b * s * k_count, cf)


def _bn_relu(x, g, bb):
    m = jnp.mean(x, axis=0, keepdims=True)
    v = jnp.mean((x - m) ** 2, axis=0, keepdims=True)
    return jax.nn.relu((x - m) * jax.lax.rsqrt(v + 1e-5) * g + bb)


def _mlp_chain(xf, w1, g1, b1, w2, g2, b2, w3, g3, b3):
    x = jnp.dot(xf, w1, preferred_element_type=jnp.float32)
    x = _bn_relu(x, g1, b1)
    x = jnp.dot(x, w2, preferred_element_type=jnp.float32)
    x = _bn_relu(x, g2, b2)
    x = jnp.dot(x, w3, preferred_element_type=jnp.float32)
    return _bn_relu(x, g3, b3)


def _mlp_body(pre_ref, g1_ref, b1_ref, w2_ref, g2_ref, b2_ref, w3_ref,
              g3_ref, b3_ref, out_ref, *, b, s_count, k_count):
    c1 = pre_ref.shape[-1]
    x = pre_ref[...].reshape(b * s_count * k_count, c1)
    x = _bn_relu(x, g1_ref[...], b1_ref[...])
    x = jnp.dot(x, w2_ref[...], preferred_element_type=jnp.float32)
    x = _bn_relu(x, g2_ref[...], b2_ref[...])
    x = jnp.dot(x, w3_ref[...], preferred_element_type=jnp.float32)
    x = _bn_relu(x, g3_ref[...], b3_ref[...])
    pooled = jnp.max(x.reshape(b * s_count, k_count, -1), axis=1)
    out_ref[...] = pooled.reshape(b, s_count, -1)


def _tail_body(obs2_ref, xyzT2_ref,
               w21_ref, g21_ref, b21_ref, w22_ref, g22_ref, b22_ref,
               w23_ref, g23_ref, b23_ref,
               w31_ref, g31_ref, b31_ref, w32_ref, g32_ref, b32_ref,
               w33_ref, g33_ref, b33_ref,
               mask_ref, out_ref, *, s2, k2, r2, k3, r3):
    obs2 = obs2_ref[...]
    b, n2, _ = obs2.shape
    x_pl = xyzT2_ref[:, 0, :]
    y_pl = xyzT2_ref[:, 1, :]
    z_pl = xyzT2_ref[:, 2, :]

    cxs, cys, czs = _fps_vec(x_pl, y_pl, z_pl, s2, n2)
    x16 = _bq_group(x_pl, y_pl, z_pl, cxs, cys, czs, obs2, r2, k2)
    x = _mlp_chain(x16, w21_ref[...], g21_ref[...], b21_ref[...],
                   w22_ref[...], g22_ref[...], b22_ref[...],
                   w23_ref[...], g23_ref[...], b23_ref[...])
    feats2 = jnp.max(x.reshape(b * s2, k2, -1), axis=1)
    feats2 = feats2.reshape(b, s2, -1)

    newxyz2 = jnp.concatenate(
        [cxs[:, :, None], cys[:, :, None], czs[:, :, None]], axis=2)
    obs3 = jnp.concatenate([newxyz2, feats2], axis=-1)
    x16 = _bq_group(cxs, cys, czs,
                    cxs[:, 0:1], cys[:, 0:1], czs[:, 0:1],
                    obs3, r3, k3)
    x = _mlp_chain(x16, w31_ref[...], g31_ref[...], b31_ref[...],
                   w32_ref[...], g32_ref[...], b32_ref[...],
                   w33_ref[...], g33_ref[...], b33_ref[...])
    pooled = jnp.max(x.reshape(b, k3, -1), axis=1)
    out_ref[...] = pooled * mask_ref[...]


def kernel(observation, mask, params):
    b, n, cf = observation.shape
    xyz_t = observation[:, :, :3].transpose(0, 2, 1)
    (npoint, radius, nsample) = (64, 1.2, 16)
    lp = params[0]
    c1 = lp[0][0].shape[0]

    fps = pl.pallas_call(
        functools.partial(_fps_body, n=n, s_count=npoint),
        out_shape=jax.ShapeDtypeStruct((b, npoint, 3), jnp.float32),
    )
    newxyz = fps(xyz_t)

    geom = pl.pallas_call(
        functools.partial(_geom_body, n=n, s_count=npoint,
                          k_count=nsample, radius=radius),
        grid=(b,),
        in_specs=[
            pl.BlockSpec((1, npoint, 3), lambda i: (i, 0, 0)),
            pl.BlockSpec((1, 3, n), lambda i: (i, 0, 0)),
            pl.BlockSpec((1, n, cf), lambda i: (i, 0, 0)),
            pl.BlockSpec((cf, c1), lambda i: (0, 0)),
        ],
        out_specs=pl.BlockSpec((1, npoint, nsample, c1),
                               lambda i: (i, 0, 0, 0)),
        out_shape=jax.ShapeDtypeStruct((b, npoint, nsample, c1),
                                       jnp.float32),
    )
    pre = geom(newxyz, xyz_t, observation, lp[0][0].T)

    c3 = lp[2][0].shape[0]
    mlp = pl.pallas_call(
        functools.partial(_mlp_body, b=b, s_count=npoint, k_count=nsample),
        out_shape=jax.ShapeDtypeStruct((b, npoint, c3), jnp.float32),
    )
    feats1 = mlp(pre,
                 lp[0][1].reshape(1, -1), lp[0][2].reshape(1, -1),
                 lp[1][0].T, lp[1][1].reshape(1, -1), lp[1][2].reshape(1, -1),
                 lp[2][0].T, lp[2][1].reshape(1, -1), lp[2][2].reshape(1, -1))

    obs2 = jnp.concatenate([newxyz, feats1], axis=-1)
    xyz_t2 = newxyz.transpose(0, 2, 1)
    lp2, lp3 = params[1], params[2]
    tail = pl.pallas_call(
        functools.partial(_tail_body, s2=16, k2=16, r2=3.6, k3=16, r3=3.6),
        out_shape=jax.ShapeDtypeStruct((b, lp3[2][0].shape[0]), jnp.float32),
    )
    y = tail(obs2, xyz_t2,
             lp2[0][0].T, lp2[0][1].reshape(1, -1), lp2[0][2].reshape(1, -1),
             lp2[1][0].T, lp2[1][1].reshape(1, -1), lp2[1][2].reshape(1, -1),
             lp2[2][0].T, lp2[2][1].reshape(1, -1), lp2[2][2].reshape(1, -1),
             lp3[0][0].T, lp3[0][1].reshape(1, -1), lp3[0][2].reshape(1, -1),
             lp3[1][0].T, lp3[1][1].reshape(1, -1), lp3[1][2].reshape(1, -1),
             lp3[2][0].T, lp3[2][1].reshape(1, -1), lp3[2][2].reshape(1, -1),
             mask)
    return (y, y)
